# baseline (speedup 1.0000x reference)
import functools

import jax
import jax.numpy as jnp
from jax.experimental import pallas as pl
from jax.experimental.pallas import tpu as pltpu

_PAD = 512
_NEG = -1e30


def _postproc_kernel(num_select, x_ref, b_ref, ts_ref, s_ref, l_ref,
                     bx_ref, or_ref):
  xv = x_ref[0]
  n_rows, n_cls = xv.shape
  n_tiles = n_rows // 8

  cm = jnp.max(xv.reshape(n_tiles, 8, n_cls), axis=1)
  m1 = jnp.max(cm.T, axis=0)[None, :]

  iota_t = jax.lax.broadcasted_iota(jnp.int32, (1, n_tiles), 1)
  iota2d = (jax.lax.broadcasted_iota(jnp.int32, (8, n_cls), 0) * 128 +
            jax.lax.broadcasted_iota(jnp.int32, (8, n_cls), 1))
  iota_r8 = jax.lax.broadcasted_iota(jnp.int32, (8, 128), 0)
  iota_l128 = jax.lax.broadcasted_iota(jnp.int32, (1, 128), 1)
  iota_out = jax.lax.broadcasted_iota(jnp.int32, (1, _PAD), 1)
  big = jnp.int32(1 << 30)

  def body(i, carry):
    m1, vals, labs, cxa, cya, wa, ha = carry
    gmax = jnp.max(m1)
    t = jnp.min(jnp.where(m1 == gmax, iota_t, big))
    tile = x_ref[0, pl.ds(t * 8, 8), :]
    pmin = jnp.min(jnp.where(tile == gmax, iota2d, big))
    r = pmin // 128
    c = pmin % 128
    n = t * 8 + r
    newtile = jnp.where(iota2d == pmin, _NEG, tile)
    x_ref[0, pl.ds(t * 8, 8), :] = newtile
    m1 = jnp.where(iota_t == t, jnp.max(newtile), m1)

    s = n // 32
    l = (n % 32) * 4
    sb = (s // 8) * 8
    btile = b_ref[0, pl.ds(sb, 8), :]
    row = jnp.sum(jnp.where(iota_r8 == (s - sb), btile, 0.0), axis=0,
                  keepdims=True)
    cx = jnp.sum(jnp.where(iota_l128 == l, row, 0.0))
    cy = jnp.sum(jnp.where(iota_l128 == l + 1, row, 0.0))
    w = jnp.sum(jnp.where(iota_l128 == l + 2, row, 0.0))
    h = jnp.sum(jnp.where(iota_l128 == l + 3, row, 0.0))

    hot = iota_out == i
    vals = jnp.where(hot, gmax, vals)
    labs = jnp.where(hot, c, labs)
    cxa = jnp.where(hot, cx, cxa)
    cya = jnp.where(hot, cy, cya)
    wa = jnp.where(hot, w, wa)
    ha = jnp.where(hot, h, ha)
    return m1, vals, labs, cxa, cya, wa, ha

  zf = jnp.zeros((1, _PAD), jnp.float32)
  zi = jnp.zeros((1, _PAD), jnp.int32)
  m1, vals, labs, cxa, cya, wa, ha = jax.lax.fori_loop(
      0, num_select, body, (m1, zf, zi, zf, zf, zf, zf))

  s_ref[0] = jax.nn.sigmoid(vals)
  l_ref[0] = labs

  x0 = cxa - 0.5 * wa
  y0 = cya - 0.5 * ha
  x1 = cxa + 0.5 * wa
  y1 = cya + 0.5 * ha
  tsrow = ts_ref[0, 0:1, :]
  img_h = jnp.sum(jnp.where(iota_l128 == 0, tsrow, 0.0))
  img_w = jnp.sum(jnp.where(iota_l128 == 1, tsrow, 0.0))
  zrow = jnp.zeros((4, _PAD), jnp.float32)
  or_ref[0] = jnp.concatenate([x0, y0, x1, y1, zrow], axis=0)
  bx_ref[0] = jnp.concatenate(
      [x0 * img_w, y0 * img_h, x1 * img_w, y1 * img_h, zrow], axis=0)


def _run(pred_logits, pred_boxes, target_sizes, num_select, interpret=False):
  b, n, c = pred_logits.shape
  boxes_flat = pred_boxes.reshape(b, (n * 4) // 128, 128)
  ts_pad = jnp.pad(target_sizes[:, None, :], ((0, 0), (0, 7), (0, 126)))

  s, l, bx, orr = pl.pallas_call(
      functools.partial(_postproc_kernel, num_select),
      grid=(b,),
      in_specs=[
          pl.BlockSpec((1, n, c), lambda i: (i, 0, 0)),
          pl.BlockSpec((1, (n * 4) // 128, 128), lambda i: (i, 0, 0)),
          pl.BlockSpec((1, 8, 128), lambda i: (i, 0, 0)),
      ],
      out_specs=[
          pl.BlockSpec((1, 1, _PAD), lambda i: (i, 0, 0)),
          pl.BlockSpec((1, 1, _PAD), lambda i: (i, 0, 0)),
          pl.BlockSpec((1, 8, _PAD), lambda i: (i, 0, 0)),
          pl.BlockSpec((1, 8, _PAD), lambda i: (i, 0, 0)),
      ],
      out_shape=[
          jax.ShapeDtypeStruct((b, 1, _PAD), jnp.float32),
          jax.ShapeDtypeStruct((b, 1, _PAD), jnp.int32),
          jax.ShapeDtypeStruct((b, 8, _PAD), jnp.float32),
          jax.ShapeDtypeStruct((b, 8, _PAD), jnp.float32),
      ],
      compiler_params=pltpu.CompilerParams(
          dimension_semantics=("parallel",)),
      interpret=interpret,
  )(pred_logits, boxes_flat, ts_pad)

  scores = s[:, 0, :num_select]
  labels = l[:, 0, :num_select]
  boxes = jnp.transpose(bx[:, :4, :num_select], (0, 2, 1))
  ori_boxes = jnp.transpose(orr[:, :4, :num_select], (0, 2, 1))
  return scores, labels, boxes, ori_boxes


def kernel(pred_logits, pred_boxes, target_sizes):
  return _run(pred_logits, pred_boxes, target_sizes, 300)


# speedup vs baseline: 1.0228x; 1.0228x over previous
import functools

import jax
import jax.numpy as jnp
from jax.experimental import pallas as pl
from jax.experimental.pallas import tpu as pltpu

_PAD = 512
_NEG = -1e30


def _postproc_kernel(num_select, x_ref, b_ref, ts_ref, s_ref, l_ref,
                     bx_ref, or_ref):
  xv = x_ref[0]
  n_rows, n_cls = xv.shape
  n_tiles = n_rows // 8
  m_rows = -(-n_tiles // 128)

  cm = jnp.max(xv.reshape(n_tiles, 8, n_cls), axis=1)
  m1f = jnp.max(cm.T, axis=0)[None, :]
  m1f = jnp.concatenate(
      [m1f, jnp.full((1, m_rows * 128 - n_tiles), _NEG, jnp.float32)], axis=1)
  m1 = m1f.reshape(m_rows, 128)

  iota_t = (jax.lax.broadcasted_iota(jnp.int32, (m_rows, 128), 0) * 128 +
            jax.lax.broadcasted_iota(jnp.int32, (m_rows, 128), 1))
  iota2d = (jax.lax.broadcasted_iota(jnp.int32, (8, n_cls), 0) * 128 +
            jax.lax.broadcasted_iota(jnp.int32, (8, n_cls), 1))
  iota_r8 = jax.lax.broadcasted_iota(jnp.int32, (8, 128), 0)
  iota_l128 = jax.lax.broadcasted_iota(jnp.int32, (1, 128), 1)
  iota_out = (jax.lax.broadcasted_iota(jnp.int32, (8, _PAD // 8), 0) *
              (_PAD // 8) +
              jax.lax.broadcasted_iota(jnp.int32, (8, _PAD // 8), 1))
  big = jnp.int32(1 << 30)

  def body(i, carry):
    m1, vals, labs, cxa, cya, wa, ha = carry
    gmax = jnp.max(m1)
    t = jnp.min(jnp.where(m1 == gmax, iota_t, big))
    tile = x_ref[0, pl.ds(t * 8, 8), :]
    pmin = jnp.min(jnp.where(tile == gmax, iota2d, big))
    r = pmin // 128
    c = pmin % 128
    n = t * 8 + r
    newtile = jnp.where(iota2d == pmin, _NEG, tile)
    x_ref[0, pl.ds(t * 8, 8), :] = newtile
    m1 = jnp.where(iota_t == t, jnp.max(newtile), m1)

    s = n // 32
    l = (n % 32) * 4
    sb = (s // 8) * 8
    btile = b_ref[0, pl.ds(sb, 8), :]
    row = jnp.sum(jnp.where(iota_r8 == (s - sb), btile, 0.0), axis=0,
                  keepdims=True)
    cx = jnp.sum(jnp.where(iota_l128 == l, row, 0.0))
    cy = jnp.sum(jnp.where(iota_l128 == l + 1, row, 0.0))
    w = jnp.sum(jnp.where(iota_l128 == l + 2, row, 0.0))
    h = jnp.sum(jnp.where(iota_l128 == l + 3, row, 0.0))

    hot = iota_out == i
    vals = jnp.where(hot, gmax, vals)
    labs = jnp.where(hot, c, labs)
    cxa = jnp.where(hot, cx, cxa)
    cya = jnp.where(hot, cy, cya)
    wa = jnp.where(hot, w, wa)
    ha = jnp.where(hot, h, ha)
    return m1, vals, labs, cxa, cya, wa, ha

  zf = jnp.zeros((8, _PAD // 8), jnp.float32)
  zi = jnp.zeros((8, _PAD // 8), jnp.int32)
  m1, vals, labs, cxa, cya, wa, ha = jax.lax.fori_loop(
      0, num_select, body, (m1, zf, zi, zf, zf, zf, zf))

  s_ref[0] = jax.nn.sigmoid(vals)
  l_ref[0] = labs

  x0 = cxa - 0.5 * wa
  y0 = cya - 0.5 * ha
  x1 = cxa + 0.5 * wa
  y1 = cya + 0.5 * ha
  tsrow = ts_ref[0, 0:1, :]
  img_h = jnp.sum(jnp.where(iota_l128 == 0, tsrow, 0.0))
  img_w = jnp.sum(jnp.where(iota_l128 == 1, tsrow, 0.0))
  or_ref[0] = jnp.stack([x0, y0, x1, y1], axis=0)
  bx_ref[0] = jnp.stack(
      [x0 * img_w, y0 * img_h, x1 * img_w, y1 * img_h], axis=0)


def _run(pred_logits, pred_boxes, target_sizes, num_select, interpret=False):
  b, n, c = pred_logits.shape
  boxes_flat = pred_boxes.reshape(b, (n * 4) // 128, 128)
  ts_pad = jnp.pad(target_sizes[:, None, :], ((0, 0), (0, 7), (0, 126)))

  s, l, bx, orr = pl.pallas_call(
      functools.partial(_postproc_kernel, num_select),
      grid=(b,),
      in_specs=[
          pl.BlockSpec((1, n, c), lambda i: (i, 0, 0)),
          pl.BlockSpec((1, (n * 4) // 128, 128), lambda i: (i, 0, 0)),
          pl.BlockSpec((1, 8, 128), lambda i: (i, 0, 0)),
      ],
      out_specs=[
          pl.BlockSpec((1, 8, _PAD // 8), lambda i: (i, 0, 0)),
          pl.BlockSpec((1, 8, _PAD // 8), lambda i: (i, 0, 0)),
          pl.BlockSpec((1, 4, 8, _PAD // 8), lambda i: (i, 0, 0, 0)),
          pl.BlockSpec((1, 4, 8, _PAD // 8), lambda i: (i, 0, 0, 0)),
      ],
      out_shape=[
          jax.ShapeDtypeStruct((b, 8, _PAD // 8), jnp.float32),
          jax.ShapeDtypeStruct((b, 8, _PAD // 8), jnp.int32),
          jax.ShapeDtypeStruct((b, 4, 8, _PAD // 8), jnp.float32),
          jax.ShapeDtypeStruct((b, 4, 8, _PAD // 8), jnp.float32),
      ],
      compiler_params=pltpu.CompilerParams(
          dimension_semantics=("parallel",)),
      interpret=interpret,
  )(pred_logits, boxes_flat, ts_pad)

  scores = s.reshape(b, _PAD)[:, :num_select]
  labels = l.reshape(b, _PAD)[:, :num_select]
  boxes = jnp.transpose(bx.reshape(b, 4, _PAD)[:, :, :num_select], (0, 2, 1))
  ori_boxes = jnp.transpose(
      orr.reshape(b, 4, _PAD)[:, :, :num_select], (0, 2, 1))
  return scores, labels, boxes, ori_boxes


def kernel(pred_logits, pred_boxes, target_sizes):
  return _run(pred_logits, pred_boxes, target_sizes, 300)


# speedup vs baseline: 1.1124x; 1.0876x over previous
import functools

import jax
import jax.numpy as jnp
from jax.experimental import pallas as pl
from jax.experimental.pallas import tpu as pltpu

_PAD = 512
_NEG = -1e30
_GRP = 2


def _postproc_kernel(num_select, grp, x_ref, b_ref, ts_ref, s_ref, l_ref,
                     bx_ref, or_ref):
  n_rows, n_cls = x_ref.shape[1], x_ref.shape[2]
  n_tiles = n_rows // 8
  m_rows = -(-n_tiles // 128)

  iota_t = (jax.lax.broadcasted_iota(jnp.int32, (m_rows, 128), 0) * 128 +
            jax.lax.broadcasted_iota(jnp.int32, (m_rows, 128), 1))
  iota2d = (jax.lax.broadcasted_iota(jnp.int32, (8, n_cls), 0) * 128 +
            jax.lax.broadcasted_iota(jnp.int32, (8, n_cls), 1))
  iota_r8 = jax.lax.broadcasted_iota(jnp.int32, (8, 128), 0)
  iota_l128 = jax.lax.broadcasted_iota(jnp.int32, (1, 128), 1)
  iota_out = (jax.lax.broadcasted_iota(jnp.int32, (8, _PAD // 8), 0) *
              (_PAD // 8) +
              jax.lax.broadcasted_iota(jnp.int32, (8, _PAD // 8), 1))
  big = jnp.int32(1 << 30)
  zf = jnp.zeros((8, _PAD // 8), jnp.float32)
  zi = jnp.zeros((8, _PAD // 8), jnp.int32)

  def init_m1(j):
    cm = jnp.max(x_ref[j].reshape(n_tiles, 8, n_cls), axis=1)
    m1f = jnp.max(cm.T, axis=0)[None, :]
    m1f = jnp.concatenate(
        [m1f, jnp.full((1, m_rows * 128 - n_tiles), _NEG, jnp.float32)],
        axis=1)
    return m1f.reshape(m_rows, 128)

  def step(j, i, part):
    m1, vals, labs, cxa, cya, wa, ha = part
    gmax = jnp.max(m1)
    t = jnp.min(jnp.where(m1 == gmax, iota_t, big))
    tile = x_ref[j, pl.ds(t * 8, 8), :]
    pmin = jnp.min(jnp.where(tile == gmax, iota2d, big))
    r = pmin // 128
    c = pmin % 128
    n = t * 8 + r
    newtile = jnp.where(iota2d == pmin, _NEG, tile)
    x_ref[j, pl.ds(t * 8, 8), :] = newtile
    m1 = jnp.where(iota_t == t, jnp.max(newtile), m1)

    s = n // 32
    l = (n % 32) * 4
    sb = (s // 8) * 8
    btile = b_ref[j, pl.ds(sb, 8), :]
    row = jnp.sum(jnp.where(iota_r8 == (s - sb), btile, 0.0), axis=0,
                  keepdims=True)
    cx = jnp.sum(jnp.where(iota_l128 == l, row, 0.0))
    cy = jnp.sum(jnp.where(iota_l128 == l + 1, row, 0.0))
    w = jnp.sum(jnp.where(iota_l128 == l + 2, row, 0.0))
    h = jnp.sum(jnp.where(iota_l128 == l + 3, row, 0.0))

    hot = iota_out == i
    vals = jnp.where(hot, gmax, vals)
    labs = jnp.where(hot, c, labs)
    cxa = jnp.where(hot, cx, cxa)
    cya = jnp.where(hot, cy, cya)
    wa = jnp.where(hot, w, wa)
    ha = jnp.where(hot, h, ha)
    return m1, vals, labs, cxa, cya, wa, ha

  def body(i, carry):
    return tuple(step(j, i, part) for j, part in enumerate(carry))

  init = tuple((init_m1(j), zf, zi, zf, zf, zf, zf) for j in range(grp))
  final = jax.lax.fori_loop(0, num_select, body, init)

  for j in range(grp):
    _, vals, labs, cxa, cya, wa, ha = final[j]
    s_ref[j] = jax.nn.sigmoid(vals)
    l_ref[j] = labs
    x0 = cxa - 0.5 * wa
    y0 = cya - 0.5 * ha
    x1 = cxa + 0.5 * wa
    y1 = cya + 0.5 * ha
    tsrow = ts_ref[j, 0:1, :]
    img_h = jnp.sum(jnp.where(iota_l128 == 0, tsrow, 0.0))
    img_w = jnp.sum(jnp.where(iota_l128 == 1, tsrow, 0.0))
    or_ref[j] = jnp.stack([x0, y0, x1, y1], axis=0)
    bx_ref[j] = jnp.stack(
        [x0 * img_w, y0 * img_h, x1 * img_w, y1 * img_h], axis=0)


def _run(pred_logits, pred_boxes, target_sizes, num_select, interpret=False):
  b, n, c = pred_logits.shape
  g = _GRP if b % _GRP == 0 else 1
  boxes_flat = pred_boxes.reshape(b, (n * 4) // 128, 128)
  ts_pad = jnp.pad(target_sizes[:, None, :], ((0, 0), (0, 7), (0, 126)))

  s, l, bx, orr = pl.pallas_call(
      functools.partial(_postproc_kernel, num_select, g),
      grid=(b // g,),
      in_specs=[
          pl.BlockSpec((g, n, c), lambda i: (i, 0, 0)),
          pl.BlockSpec((g, (n * 4) // 128, 128), lambda i: (i, 0, 0)),
          pl.BlockSpec((g, 8, 128), lambda i: (i, 0, 0)),
      ],
      out_specs=[
          pl.BlockSpec((g, 8, _PAD // 8), lambda i: (i, 0, 0)),
          pl.BlockSpec((g, 8, _PAD // 8), lambda i: (i, 0, 0)),
          pl.BlockSpec((g, 4, 8, _PAD // 8), lambda i: (i, 0, 0, 0)),
          pl.BlockSpec((g, 4, 8, _PAD // 8), lambda i: (i, 0, 0, 0)),
      ],
      out_shape=[
          jax.ShapeDtypeStruct((b, 8, _PAD // 8), jnp.float32),
          jax.ShapeDtypeStruct((b, 8, _PAD // 8), jnp.int32),
          jax.ShapeDtypeStruct((b, 4, 8, _PAD // 8), jnp.float32),
          jax.ShapeDtypeStruct((b, 4, 8, _PAD // 8), jnp.float32),
      ],
      compiler_params=pltpu.CompilerParams(
          dimension_semantics=("parallel",)),
      interpret=interpret,
  )(pred_logits, boxes_flat, ts_pad)

  scores = s.reshape(b, _PAD)[:, :num_select]
  labels = l.reshape(b, _PAD)[:, :num_select]
  boxes = jnp.transpose(bx.reshape(b, 4, _PAD)[:, :, :num_select], (0, 2, 1))
  ori_boxes = jnp.transpose(
      orr.reshape(b, 4, _PAD)[:, :, :num_select], (0, 2, 1))
  return scores, labels, boxes, ori_boxes


def kernel(pred_logits, pred_boxes, target_sizes):
  return _run(pred_logits, pred_boxes, target_sizes, 300)
